# baseline (speedup 1.0000x reference)
import functools

import jax
import jax.numpy as jnp
from jax import lax
from jax.experimental import pallas as pl
from jax.experimental.pallas import tpu as pltpu
from jax.experimental.pallas import tpu_sc as plsc

EMBED_DIM = 32
BATCH = 16384
VOCAB = 1000002

_info = plsc.get_sparse_core_info()
_NC, _NS = _info.num_cores, _info.num_subcores
_BPW = BATCH // _NS
_NGRP = _BPW // 16
_WORDS = EMBED_DIM * _BPW

_ROW_STRIDE = -(-VOCAB // 8) * 8


def _gather_body(uid_hbm, iid_hbm, utab_hbm, itab_hbm, out_hbm,
                 idx_v, off_v, buf, sem):
    core = lax.axis_index("c")
    sid = lax.axis_index("s")
    base = sid * _BPW

    def tower(ids_hbm, tab_hbm, t):
        pltpu.sync_copy(ids_hbm.at[pl.ds(base, _BPW)], idx_v)
        flat = tab_hbm.at[0]

        @pl.loop(0, _NGRP)
        def _offsets(g):
            b0 = g * 16
            vr = idx_v[pl.ds(b0, 16)]
            for c in range(EMBED_DIM):
                off_v[pl.ds(c * _BPW + b0, 16)] = vr + c * _ROW_STRIDE

        @pl.loop(0, _WORDS // 128)
        def _issue(k):
            sl = pl.ds(k * 128, 128)
            pltpu.async_copy(flat.at[off_v.at[sl]], buf.at[sl], sem)

        pltpu.make_async_copy(flat.at[pl.ds(0, _WORDS)], buf, sem).wait()

        for c in range(EMBED_DIM):
            pltpu.sync_copy(
                buf.at[pl.ds(c * _BPW, _BPW)],
                out_hbm.at[pl.ds((t * EMBED_DIM + c) * BATCH + base, _BPW)])

    @pl.when(core == 0)
    def _user():
        tower(uid_hbm, utab_hbm, 0)

    @pl.when(core == 1)
    def _item():
        tower(iid_hbm, itab_hbm, 1)


_mesh = plsc.VectorSubcoreMesh(core_axis_name="c", subcore_axis_name="s")

_gather = functools.partial(
    pl.kernel,
    mesh=_mesh,
    out_type=jax.ShapeDtypeStruct((2 * EMBED_DIM * BATCH,), jnp.float32),
    scratch_types=[
        pltpu.VMEM((_BPW,), jnp.int32),
        pltpu.VMEM((_WORDS,), jnp.int32),
        pltpu.VMEM((_WORDS,), jnp.float32),
        pltpu.SemaphoreType.DMA,
    ],
    compiler_params=pltpu.CompilerParams(
        use_tc_tiling_on_sc=False,
        disable_bounds_checks=True,
    ),
)(_gather_body)


@jax.jit
def kernel(user_ids, item_ids, user_table, item_table):
    out = _gather(user_ids.astype(jnp.int32), item_ids.astype(jnp.int32),
                  user_table.T, item_table.T)
    return out.reshape(2, EMBED_DIM, BATCH).transpose(0, 2, 1)


# speedup vs baseline: 2.5520x; 2.5520x over previous
import functools

import jax
import jax.numpy as jnp
from jax import lax
from jax.experimental import pallas as pl
from jax.experimental.pallas import tpu as pltpu
from jax.experimental.pallas import tpu_sc as plsc

EMBED_DIM = 32
BATCH = 16384
VOCAB = 1000002

_info = plsc.get_sparse_core_info()
_NC, _NS = _info.num_cores, _info.num_subcores
_NW = _NC * _NS
_BPW = BATCH // _NW
_CHUNK = 128

_TBLK = 1024
_TGRID = -(-VOCAB // _TBLK)


def _transpose_body(x_ref, o_ref):
    o_ref[...] = x_ref[...].T


_transpose = pl.pallas_call(
    _transpose_body,
    grid=(_TGRID,),
    in_specs=[pl.BlockSpec((EMBED_DIM, _TBLK), lambda j: (0, j))],
    out_specs=pl.BlockSpec((_TBLK, EMBED_DIM), lambda j: (j, 0)),
    out_shape=jax.ShapeDtypeStruct((VOCAB, EMBED_DIM), jnp.float32),
)


def _gather_body(uid_hbm, iid_hbm, utab_hbm, itab_hbm, out_hbm,
                 idx_u, idx_i, rows_u, rows_i, sem_u, sem_i):
    wid = lax.axis_index("c") * _NS + lax.axis_index("s")
    base = wid * _BPW
    pltpu.sync_copy(uid_hbm.at[pl.ds(base, _BPW)], idx_u)
    pltpu.sync_copy(iid_hbm.at[pl.ds(base, _BPW)], idx_i)
    for j in range(_BPW // _CHUNK):
        sl = pl.ds(j * _CHUNK, _CHUNK)
        pltpu.async_copy(utab_hbm.at[idx_u.at[sl]], rows_u.at[sl], sem_u)
        pltpu.async_copy(itab_hbm.at[idx_i.at[sl]], rows_i.at[sl], sem_i)
    pltpu.make_async_copy(utab_hbm.at[pl.ds(0, _BPW)], rows_u, sem_u).wait()
    pltpu.make_async_copy(itab_hbm.at[pl.ds(0, _BPW)], rows_i, sem_i).wait()
    pltpu.sync_copy(rows_u, out_hbm.at[pl.ds(base, _BPW)])
    pltpu.sync_copy(rows_i, out_hbm.at[pl.ds(BATCH + base, _BPW)])


_mesh = plsc.VectorSubcoreMesh(core_axis_name="c", subcore_axis_name="s")

_gather = functools.partial(
    pl.kernel,
    mesh=_mesh,
    out_type=jax.ShapeDtypeStruct((2 * BATCH, EMBED_DIM), jnp.float32),
    scratch_types=[
        pltpu.VMEM((_BPW,), jnp.int32),
        pltpu.VMEM((_BPW,), jnp.int32),
        pltpu.VMEM((_BPW, EMBED_DIM), jnp.float32),
        pltpu.VMEM((_BPW, EMBED_DIM), jnp.float32),
        pltpu.SemaphoreType.DMA,
        pltpu.SemaphoreType.DMA,
    ],
    compiler_params=pltpu.CompilerParams(use_tc_tiling_on_sc=False),
)(_gather_body)


@jax.jit
def kernel(user_ids, item_ids, user_table, item_table):
    ut = _transpose(user_table.T)
    it = _transpose(item_table.T)
    out = _gather(user_ids.astype(jnp.int32), item_ids.astype(jnp.int32),
                  ut, it)
    return out.reshape(2, BATCH, EMBED_DIM)


# speedup vs baseline: 5.7544x; 2.2548x over previous
import functools

import jax
import jax.numpy as jnp
from jax import lax
from jax.experimental import pallas as pl
from jax.experimental.pallas import tpu as pltpu
from jax.experimental.pallas import tpu_sc as plsc

EMBED_DIM = 32
BATCH = 16384

_info = plsc.get_sparse_core_info()
_NC, _NS = _info.num_cores, _info.num_subcores
_NW = _NC * _NS
_BPW = BATCH // _NW
_CHUNK = 128
_NCHUNK = _BPW // _CHUNK


def _gather_body(uid_hbm, iid_hbm, utab_hbm, itab_hbm, out_hbm,
                 idx_u, idx_i, rows_u, rows_i, sem):
    wid = lax.axis_index("s") * _NC + lax.axis_index("c")
    base = wid * _BPW
    pltpu.sync_copy(uid_hbm.at[pl.ds(base, _BPW)], idx_u)
    pltpu.sync_copy(iid_hbm.at[pl.ds(base, _BPW)], idx_i)
    copies = []
    for j in range(_NCHUNK):
        sl = pl.ds(j * _CHUNK, _CHUNK)
        copies.append(pltpu.async_copy(utab_hbm.at[idx_u.at[sl]], rows_u.at[sl], sem))
        copies.append(pltpu.async_copy(itab_hbm.at[idx_i.at[sl]], rows_i.at[sl], sem))
    for c in copies:
        c.wait()
    pltpu.sync_copy(rows_u, out_hbm.at[pl.ds(base, _BPW)])
    pltpu.sync_copy(rows_i, out_hbm.at[pl.ds(BATCH + base, _BPW)])


_mesh = plsc.VectorSubcoreMesh(core_axis_name="c", subcore_axis_name="s")

_gather = functools.partial(
    pl.kernel,
    mesh=_mesh,
    out_type=jax.ShapeDtypeStruct((2 * BATCH, EMBED_DIM), jnp.float32),
    scratch_types=[
        pltpu.VMEM((_BPW,), jnp.int32),
        pltpu.VMEM((_BPW,), jnp.int32),
        pltpu.VMEM((_BPW, EMBED_DIM), jnp.float32),
        pltpu.VMEM((_BPW, EMBED_DIM), jnp.float32),
        pltpu.SemaphoreType.DMA,
    ],
    compiler_params=pltpu.CompilerParams(use_tc_tiling_on_sc=False),
)(_gather_body)


@jax.jit
def kernel(user_ids, item_ids, user_table, item_table):
    out = _gather(user_ids.astype(jnp.int32), item_ids.astype(jnp.int32),
                  user_table, item_table)
    return out.reshape(2, BATCH, EMBED_DIM)
